# Initial kernel scaffold; baseline (speedup 1.0000x reference)
import jax
import jax.numpy as jnp
from jax.experimental import pallas as pl


def kernel(input_one, input_two, weight):
    raise NotImplementedError("write your pallas kernel here")


# speedup vs baseline: 9.2645x; 9.2645x over previous
import functools

import jax
import jax.numpy as jnp
from jax import lax
from jax.experimental import pallas as pl
from jax.experimental.pallas import tpu as pltpu
from jax.experimental.pallas import tpu_sc as plsc

NUM_EMB = 100000
DIM = 64
BATCH = 1024

NUM_CORES = 2
NUM_SUBCORES = 16
NW = NUM_CORES * NUM_SUBCORES
BPW = BATCH // NW
L = 16


def _sc_body(i1_hbm, i2_hbm, w_hbm, out_hbm, i1_v, i2_v, r1_v, r2_v, sem):
    wid = lax.axis_index("s") * NUM_CORES + lax.axis_index("c")
    base = wid * BPW

    pltpu.sync_copy(i1_hbm.at[pl.ds(base, BPW)], i1_v)
    pltpu.sync_copy(i2_hbm.at[pl.ds(base, BPW)], i2_v)

    cp1 = pltpu.async_copy(w_hbm.at[i1_v], r1_v, sem)
    cp2 = pltpu.async_copy(w_hbm.at[i2_v], r2_v, sem)
    cp1.wait()
    cp2.wait()

    for g in range(BPW // L):
        a = i1_v[pl.ds(g * L, L)]
        b = i2_v[pl.ds(g * L, L)]
        sv = jnp.where(a != b, jnp.float32(1.0), jnp.float32(0.0))
        for j in range(L):
            r = g * L + j
            s = jnp.broadcast_to(sv[j], (L,))
            for c in range(DIM // L):
                sl = pl.ds(c * L, L)
                r1_v[r, sl] = r1_v[r, sl] + s * r2_v[r, sl]

    pltpu.sync_copy(r1_v, out_hbm.at[pl.ds(base, BPW)])


_two_hot_sc = functools.partial(
    pl.kernel,
    out_type=jax.ShapeDtypeStruct((BATCH, DIM), jnp.float32),
    mesh=plsc.VectorSubcoreMesh(core_axis_name="c", subcore_axis_name="s"),
    compiler_params=pltpu.CompilerParams(use_tc_tiling_on_sc=False),
    scratch_types=[
        pltpu.VMEM((BPW,), jnp.int32),
        pltpu.VMEM((BPW,), jnp.int32),
        pltpu.VMEM((BPW, DIM), jnp.float32),
        pltpu.VMEM((BPW, DIM), jnp.float32),
        pltpu.SemaphoreType.DMA,
    ],
)(_sc_body)


@jax.jit
def kernel(input_one, input_two, weight):
    i1 = input_one.astype(jnp.int32)
    i2 = input_two.astype(jnp.int32)
    return _two_hot_sc(i1, i2, weight)


# speedup vs baseline: 13.5384x; 1.4613x over previous
import functools

import jax
import jax.numpy as jnp
from jax import lax
from jax.experimental import pallas as pl
from jax.experimental.pallas import tpu as pltpu
from jax.experimental.pallas import tpu_sc as plsc

NUM_EMB = 100000
DIM = 64
BATCH = 1024

NUM_CORES = 2
NUM_SUBCORES = 16
NW = NUM_CORES * NUM_SUBCORES
BPW = BATCH // NW
L = 16


def _sc_body(i1_hbm, i2_hbm, w_hbm, out_hbm, i1_v, i2_v, r1_v, r2_v, sem):
    wid = lax.axis_index("s") * NUM_CORES + lax.axis_index("c")
    base = wid * BPW

    pltpu.sync_copy(i1_hbm.at[pl.ds(base, BPW)], i1_v)
    pltpu.sync_copy(i2_hbm.at[pl.ds(base, BPW)], i2_v)

    copies = []
    scales = []
    for g in range(BPW // L):
        a = i1_v[pl.ds(g * L, L)]
        b = i2_v[pl.ds(g * L, L)]
        scales.append(jnp.where(a != b, jnp.float32(1.0), jnp.float32(0.0)))
        for j in range(L):
            r = g * L + j
            copies.append(pltpu.async_copy(
                w_hbm.at[pl.ds(a[j], 1), :], r1_v.at[pl.ds(r, 1), :], sem))
            copies.append(pltpu.async_copy(
                w_hbm.at[pl.ds(b[j], 1), :], r2_v.at[pl.ds(r, 1), :], sem))
    for cp in copies:
        cp.wait()

    for g in range(BPW // L):
        sv = scales[g]
        for j in range(L):
            r = g * L + j
            s = jnp.broadcast_to(sv[j], (L,))
            for c in range(DIM // L):
                sl = pl.ds(c * L, L)
                r1_v[r, sl] = r1_v[r, sl] + s * r2_v[r, sl]

    pltpu.sync_copy(r1_v, out_hbm.at[pl.ds(base, BPW), :])


_two_hot_sc = functools.partial(
    pl.kernel,
    out_type=jax.ShapeDtypeStruct((BATCH, DIM), jnp.float32),
    mesh=plsc.VectorSubcoreMesh(core_axis_name="c", subcore_axis_name="s"),
    scratch_types=[
        pltpu.VMEM((BPW,), jnp.int32),
        pltpu.VMEM((BPW,), jnp.int32),
        pltpu.VMEM((BPW, DIM), jnp.float32),
        pltpu.VMEM((BPW, DIM), jnp.float32),
        pltpu.SemaphoreType.DMA,
    ],
)(_sc_body)


@jax.jit
def kernel(input_one, input_two, weight):
    i1 = input_one.astype(jnp.int32)
    i2 = input_two.astype(jnp.int32)
    return _two_hot_sc(i1, i2, weight)


# speedup vs baseline: 17.7735x; 1.3128x over previous
import functools

import jax
import jax.numpy as jnp
from jax import lax
from jax.experimental import pallas as pl
from jax.experimental.pallas import tpu as pltpu
from jax.experimental.pallas import tpu_sc as plsc

NUM_EMB = 100000
DIM = 64
BATCH = 1024

NUM_CORES = 2
NUM_SUBCORES = 16
L = 16
NCHUNK = 8
CW = 12544
CW_LAST = NUM_EMB - (NCHUNK - 1) * CW
CW_LAST_DMA = ((CW_LAST + 127) // 128) * 128
DPC = DIM // NUM_CORES
NGRP = DPC // 8
ROWS_PER_TILE = DPC // NUM_SUBCORES


def _sc_body(i1_hbm, i2_hbm, wt_hbm, out_hbm,
             i1_v, i2_v, scale_v, slab_v, part_v,
             fb0, fb1, fb2, fb3, out_v, shared, sem):
    s = lax.axis_index("s")
    c = lax.axis_index("c")
    glocal = s % NGRP
    owner = s // NGRP
    dbase = DPC * c + 8 * glocal

    pltpu.sync_copy(i1_hbm.at[pl.ds(0, BATCH)], i1_v)
    pltpu.sync_copy(i2_hbm.at[pl.ds(0, BATCH)], i2_v)

    iota = lax.iota(jnp.int32, L)

    def _prep(t, carry):
        a = i1_v[pl.ds(t * L, L)]
        b = i2_v[pl.ds(t * L, L)]
        scale_v[pl.ds(t * L, L)] = jnp.where(
            a != b, jnp.float32(1.0), jnp.float32(0.0))
        z = jnp.zeros((L,), jnp.float32)
        for dl in range(8):
            part_v[dl, pl.ds(t * L, L)] = z
        return carry
    lax.fori_loop(0, BATCH // L, _prep, 0)

    for r in range(2):
        q = owner + NGRP * r
        cbase = pl.multiple_of(q * CW, 128)
        w = jnp.where(q == NCHUNK - 1, CW_LAST, CW).astype(jnp.int32)

        @pl.when(q == NCHUNK - 1)
        def _():
            pltpu.sync_copy(
                wt_hbm.at[pl.ds(dbase, 8), pl.ds(cbase, CW_LAST_DMA)],
                slab_v.at[:, pl.ds(0, CW_LAST_DMA)])

        @pl.when(q != NCHUNK - 1)
        def _():
            pltpu.sync_copy(
                wt_hbm.at[pl.ds(dbase, 8), pl.ds(cbase, CW)], slab_v)

        def _accum(t, carry, idx_ref, scaled):
            idx = idx_ref[pl.ds(t * L, L)]
            local = idx - cbase
            m = (local >= 0) & (local < w)
            bvec = t * L + iota
            if scaled:
                scl = scale_v[pl.ds(t * L, L)]
            for dl in range(8):
                dsplat = jnp.full((L,), dl, jnp.int32)
                v = plsc.load_gather(slab_v, [dsplat, local], mask=m)
                if scaled:
                    v = v * scl
                plsc.addupdate_scatter(part_v, [dsplat, bvec], v, mask=m)
            return carry

        lax.fori_loop(0, BATCH // L,
                      functools.partial(_accum, idx_ref=i1_v, scaled=False), 0)
        lax.fori_loop(0, BATCH // L,
                      functools.partial(_accum, idx_ref=i2_v, scaled=True), 0)

    pltpu.sync_copy(part_v, shared.at[owner, pl.ds(8 * glocal, 8), :])
    plsc.subcore_barrier()

    rbase = ROWS_PER_TILE * s
    pltpu.sync_copy(shared.at[0, pl.ds(rbase, ROWS_PER_TILE), :], fb0)
    pltpu.sync_copy(shared.at[1, pl.ds(rbase, ROWS_PER_TILE), :], fb1)
    pltpu.sync_copy(shared.at[2, pl.ds(rbase, ROWS_PER_TILE), :], fb2)
    pltpu.sync_copy(shared.at[3, pl.ds(rbase, ROWS_PER_TILE), :], fb3)

    def _reduce(t, carry):
        for row in range(ROWS_PER_TILE):
            sl = pl.ds(t * L, L)
            out_v[row, sl] = ((fb0[row, sl] + fb1[row, sl])
                              + (fb2[row, sl] + fb3[row, sl]))
        return carry
    lax.fori_loop(0, BATCH // L, _reduce, 0)

    pltpu.sync_copy(out_v, out_hbm.at[pl.ds(DPC * c + rbase, ROWS_PER_TILE), :])


_two_hot_sc = functools.partial(
    pl.kernel,
    out_type=jax.ShapeDtypeStruct((DIM, BATCH), jnp.float32),
    mesh=plsc.VectorSubcoreMesh(core_axis_name="c", subcore_axis_name="s"),
    compiler_params=pltpu.CompilerParams(needs_layout_passes=False),
    scratch_types=[
        pltpu.VMEM((BATCH,), jnp.int32),
        pltpu.VMEM((BATCH,), jnp.int32),
        pltpu.VMEM((BATCH,), jnp.float32),
        pltpu.VMEM((8, CW), jnp.float32),
        pltpu.VMEM((8, BATCH), jnp.float32),
        pltpu.VMEM((ROWS_PER_TILE, BATCH), jnp.float32),
        pltpu.VMEM((ROWS_PER_TILE, BATCH), jnp.float32),
        pltpu.VMEM((ROWS_PER_TILE, BATCH), jnp.float32),
        pltpu.VMEM((ROWS_PER_TILE, BATCH), jnp.float32),
        pltpu.VMEM((ROWS_PER_TILE, BATCH), jnp.float32),
        pltpu.VMEM_SHARED((NGRP, DPC, BATCH), jnp.float32),
        pltpu.SemaphoreType.DMA,
    ],
)(_sc_body)


@jax.jit
def kernel(input_one, input_two, weight):
    i1 = input_one.astype(jnp.int32)
    i2 = input_two.astype(jnp.int32)
    return _two_hot_sc(i1, i2, weight.T).T
